# baseline (speedup 1.0000x reference)
import functools

import jax
import jax.numpy as jnp
import numpy as np
from jax import lax
from jax.experimental import pallas as pl
from jax.experimental.pallas import tpu as pltpu
from jax.experimental.pallas import tpu_sc as plsc

K = 64
B = 128
N = 32768
L = 16
NW = 64
WV = 32
NG = NW * L
NSUB = 32
RPW = B // NSUB
NVEC = N // L
CAND = K * WV

U32 = jnp.uint32
I32 = jnp.int32
F32 = jnp.float32

_TOPBIT = np.uint32(0x80000000)
_ALLONE = np.uint32(0xFFFFFFFF)


def _vkeys(v):
    bits = lax.bitcast_convert_type(v, U32)
    neg = bits >= _TOPBIT
    return jnp.where(neg, ~bits, bits | _TOPBIT)


def _splat_sum(v):
    tot_last = plsc.cumsum(v)
    tot_first = lax.rev(tot_last, (0,))
    first = (lax.iota(I32, L) == 0).astype(v.dtype)
    return plsc.cumsum(tot_first * first)


def _popc(m):
    return plsc.all_reduce_population_count(m)


def _sc_body(x_hbm, o_hbm, row_v, gk_v, sel_v, cand_v, candk_v):
    wid = lax.axis_index("s") * 2 + lax.axis_index("c")
    lanes = lax.iota(I32, L)

    def do_row(r, _):
        row = wid * RPW + r
        base = row * N
        pltpu.sync_copy(x_hbm.at[pl.ds(base, N)], row_v)

        def win_body(w, racc):
            gmax = None
            for i in range(WV):
                v = row_v[pl.ds((w * WV + i) * L, L)]
                racc = racc + jnp.maximum(v, 0.0)
                gmax = v if gmax is None else jnp.maximum(gmax, v)
            gk_v[pl.ds(w * L, L)] = _vkeys(gmax)
            return racc

        racc = lax.fori_loop(0, NW, win_body, jnp.zeros((L,), F32))
        mag = _splat_sum(racc)

        zero_u = jnp.zeros((L,), U32)

        def csearch(_, lohi):
            lo, hi = lohi
            mid = lo + ((hi - lo) >> U32(1))

            def cnt_body(j, acc):
                g = gk_v[pl.ds(j * L, L)]
                return acc + _popc(g > mid)

            cnt = lax.fori_loop(0, NW, cnt_body, jnp.zeros((L,), I32))
            go_up = cnt >= K
            return (jnp.where(go_up, mid + U32(1), lo), jnp.where(go_up, hi, mid))

        ck, _unused = lax.fori_loop(0, 32, csearch, (zero_u, zero_u + _ALLONE))

        for j in range(4):
            sel_v[pl.ds(j * L, L)] = jnp.zeros((L,), I32)

        def compact(j, tot):
            g = gk_v[pl.ds(j * L, L)]
            m = g > ck
            inc = jnp.where(m, 1, 0).astype(I32)
            pos = plsc.cumsum(inc) - 1 + tot
            gids = j * L + lanes
            plsc.store_scatter(sel_v, [pos], gids, mask=m)
            return tot + _popc(m)

        n_strict = lax.fori_loop(0, NW, compact, jnp.zeros((L,), I32))

        neg_inf = jnp.full((L,), -jnp.inf, F32)
        for sv in range(4):
            g = sel_v[pl.ds(sv * L, L)]
            w = lax.shift_right_logical(g, 4)
            l = lax.bitwise_and(g, 15)
            gbase = w * (WV * L) + l
            valid = (sv * L + lanes) < n_strict
            for i in range(WV):
                vals = plsc.load_gather(row_v, [gbase + i * L])
                vals = jnp.where(valid, vals, neg_inf)
                slot = (sv * WV + i) * L
                cand_v[pl.ds(slot, L)] = vals
                candk_v[pl.ds(slot, L)] = _vkeys(vals)

        def tsearch(_, lohi):
            lo, hi = lohi
            mid = lo + ((hi - lo) >> U32(1))

            def cnt_body(j, acc):
                ckv = candk_v[pl.ds(j * L, L)]
                return acc + _popc(ckv > mid)

            cnt = lax.fori_loop(0, CAND // L, cnt_body, jnp.zeros((L,), I32))
            go_up = cnt >= K
            return (jnp.where(go_up, mid + U32(1), lo), jnp.where(go_up, hi, mid))

        tk, _unused2 = lax.fori_loop(0, 32, tsearch, (ck, zero_u + _ALLONE))

        def sum_body(j, accs):
            sacc, qacc = accs
            cv = cand_v[pl.ds(j * L, L)]
            ckv = candk_v[pl.ds(j * L, L)]
            above = ckv > tk
            sacc = sacc + jnp.where(above, cv, 0.0)
            qacc = qacc + _popc(above)
            return sacc, qacc

        sacc, qacc = lax.fori_loop(
            0, CAND // L, sum_body, (jnp.zeros((L,), F32), jnp.zeros((L,), I32))
        )
        s_above = _splat_sum(sacc)
        q = qacc.astype(F32)

        tbits = jnp.where(tk >= _TOPBIT, tk ^ _TOPBIT, ~tk)
        t = lax.bitcast_convert_type(tbits, F32)

        magk = s_above + t * (F32(K) - q)
        scale = jnp.where(mag > 0.0, magk / mag, 0.0)

        def out_body(j, _unused3):
            for i in range(L):
                off = (j * L + i) * L
                v = row_v[pl.ds(off, L)]
                row_v[pl.ds(off, L)] = jnp.maximum(v, 0.0) * scale
            return 0

        lax.fori_loop(0, NVEC // L, out_body, 0)
        pltpu.sync_copy(row_v, o_hbm.at[pl.ds(base, N)])
        return 0

    lax.fori_loop(0, RPW, do_row, 0)


@jax.jit
def kernel(x):
    b, n = x.shape
    xf = x.reshape(b * n)
    mesh = plsc.VectorSubcoreMesh(core_axis_name="c", subcore_axis_name="s")
    run = functools.partial(
        pl.kernel,
        mesh=mesh,
        compiler_params=pltpu.CompilerParams(needs_layout_passes=False),
        out_type=jax.ShapeDtypeStruct((b * n,), F32),
        scratch_types=[
            pltpu.VMEM((N,), F32),
            pltpu.VMEM((NG,), U32),
            pltpu.VMEM((K,), I32),
            pltpu.VMEM((CAND,), F32),
            pltpu.VMEM((CAND,), U32),
        ],
    )(_sc_body)
    return run(xf).reshape(b, n)


# speedup vs baseline: 1.7206x; 1.7206x over previous
import functools

import jax
import jax.numpy as jnp
import numpy as np
from jax import lax
from jax.experimental import pallas as pl
from jax.experimental.pallas import tpu as pltpu
from jax.experimental.pallas import tpu_sc as plsc

K = 64
B = 128
N = 32768
L = 16
NW = 64
WV = 32
NG = NW * L
NSUB = 32
RPW = B // NSUB
NVEC = N // L
GKC = 1040
EKC = 2048

I32 = jnp.int32
F32 = jnp.float32
IMIN = np.int32(-2147483648)


def _ikeys(v):
    y = lax.bitcast_convert_type(v, I32)
    return jnp.where(y < 0, IMIN - y, y)


def _ivals(k):
    return lax.bitcast_convert_type(jnp.where(k < 0, IMIN - k, k), F32)


def _splat_sum_f32(v):
    tot_last = plsc.cumsum(v)
    tot_first = lax.rev(tot_last, (0,))
    first = (lax.iota(I32, L) == 0).astype(v.dtype)
    return plsc.cumsum(tot_first * first)


def _popc(m):
    return plsc.all_reduce_population_count(m)


def _bsearch(buf, nv, lo0, hi0):
    def cond(c):
        lo, hi = c
        return lo < hi

    def body(c):
        lo, hi = c
        mid = (lo & hi) + ((lo ^ hi) >> 1)
        mids = jnp.broadcast_to(mid, (L,))

        def cnt_body(j, acc):
            g = buf[pl.ds(j * L, L)]
            return acc + jnp.where(g > mids, 1, 0).astype(I32)

        cnt = jnp.sum(lax.fori_loop(0, nv, cnt_body, jnp.zeros((L,), I32)))
        go_up = cnt >= K
        return (jnp.where(go_up, mid + 1, lo), jnp.where(go_up, hi, mid))

    lo, _ = lax.while_loop(cond, body, (lo0, hi0))
    return lo


def _sc_body(x_hbm, o_hbm, row_v, gk_v, sel_v, gkc_v, gidc_v, ekc_v):
    wid = lax.axis_index("s") * 2 + lax.axis_index("c")
    lanes = lax.iota(I32, L)
    zero_i = jnp.zeros((L,), I32)

    def do_row(r, _):
        row = wid * RPW + r
        base = row * N
        pltpu.sync_copy(x_hbm.at[pl.ds(base, N)], row_v)

        racc = jnp.zeros((L,), F32)
        sgmax = []
        for j2 in range(4):

            def win_body(w2, carry, j2=j2):
                racc, w2acc = carry
                w = j2 * 16 + w2
                gmax = None
                for i in range(WV):
                    v = row_v[pl.ds((w * WV + i) * L, L)]
                    racc = racc + jnp.maximum(v, 0.0)
                    gmax = v if gmax is None else jnp.maximum(gmax, v)
                gkey = _ikeys(gmax)
                gk_v[pl.ds(w * L, L)] = gkey
                return racc, jnp.maximum(w2acc, gkey)

            racc, w2acc = lax.fori_loop(
                0, 16, win_body, (racc, jnp.full((L,), IMIN, I32))
            )
            sgmax.append(w2acc)

        mag = _splat_sum_f32(racc)
        c2 = jnp.min(jnp.minimum(jnp.minimum(sgmax[0], sgmax[1]),
                                 jnp.minimum(sgmax[2], sgmax[3])))
        hi0 = jnp.max(jnp.maximum(jnp.maximum(sgmax[0], sgmax[1]),
                                  jnp.maximum(sgmax[2], sgmax[3])))
        c2s = jnp.broadcast_to(c2, (L,))

        def compact_g(j, tot):
            g = gk_v[pl.ds(j * L, L)]
            m = g > c2s
            inc = jnp.where(m, 1, 0).astype(I32)
            pos = plsc.cumsum(inc) - 1 + tot
            plsc.store_scatter(gkc_v, [pos], g, mask=m)
            plsc.store_scatter(gidc_v, [pos], j * L + lanes, mask=m)
            return tot + _popc(m)

        totg = lax.fori_loop(0, NW, compact_g, zero_i)
        plsc.store_scatter(gkc_v, [totg + lanes], jnp.full((L,), IMIN, I32))
        totgs = jnp.max(totg)
        nvg = (totgs + 15) >> 4

        ck = _bsearch(gkc_v, nvg, c2, hi0)
        cks = jnp.broadcast_to(ck, (L,))

        for j in range(4):
            sel_v[pl.ds(j * L, L)] = zero_i

        def strictc(j, tot):
            g = gkc_v[pl.ds(j * L, L)]
            gid = gidc_v[pl.ds(j * L, L)]
            m = g > cks
            inc = jnp.where(m, 1, 0).astype(I32)
            pos = plsc.cumsum(inc) - 1 + tot
            plsc.store_scatter(sel_v, [pos], gid, mask=m)
            return tot + _popc(m)

        n_strict = lax.fori_loop(0, nvg, strictc, zero_i)

        q0 = zero_i
        for sv in range(4):
            g = sel_v[pl.ds(sv * L, L)]
            w = lax.shift_right_logical(g, 4)
            l = lax.bitwise_and(g, 15)
            gbase = w * (WV * L) + l
            valid = (sv * L + lanes) < n_strict
            for i in range(WV):
                vals = plsc.load_gather(row_v, [gbase + i * L])
                kk = _ikeys(vals)
                m = (kk > cks) & valid
                inc = jnp.where(m, 1, 0).astype(I32)
                pos = plsc.cumsum(inc) - 1 + q0
                plsc.store_scatter(ekc_v, [pos], kk, mask=m)
                q0 = q0 + _popc(m)

        plsc.store_scatter(ekc_v, [q0 + lanes], jnp.full((L,), IMIN, I32))
        q0s = jnp.max(q0)
        nvq = (q0s + 15) >> 4

        tk = _bsearch(ekc_v, nvq, ck, hi0)
        tks = jnp.broadcast_to(tk, (L,))

        def sum_body(j, accs):
            sacc, qacc = accs
            kk = ekc_v[pl.ds(j * L, L)]
            above = kk > tks
            sacc = sacc + jnp.where(above, _ivals(kk), 0.0)
            qacc = qacc + jnp.where(above, 1.0, 0.0)
            return sacc, qacc

        sacc, qacc = lax.fori_loop(
            0, nvq, sum_body, (jnp.zeros((L,), F32), jnp.zeros((L,), F32))
        )
        s_above = _splat_sum_f32(sacc)
        q = _splat_sum_f32(qacc)
        t = _ivals(tks)
        magk = s_above + t * (F32(K) - q)
        scale = jnp.where(mag > 0.0, magk / mag, 0.0)

        def out_body(j, _unused):
            for i in range(L):
                off = (j * L + i) * L
                v = row_v[pl.ds(off, L)]
                row_v[pl.ds(off, L)] = jnp.maximum(v, 0.0) * scale
            return 0

        lax.fori_loop(0, NVEC // L, out_body, 0)
        pltpu.sync_copy(row_v, o_hbm.at[pl.ds(base, N)])
        return 0

    lax.fori_loop(0, RPW, do_row, 0)


@jax.jit
def kernel(x):
    b, n = x.shape
    xf = x.reshape(b * n)
    mesh = plsc.VectorSubcoreMesh(core_axis_name="c", subcore_axis_name="s")
    run = functools.partial(
        pl.kernel,
        mesh=mesh,
        compiler_params=pltpu.CompilerParams(needs_layout_passes=False),
        out_type=jax.ShapeDtypeStruct((b * n,), F32),
        scratch_types=[
            pltpu.VMEM((N,), F32),
            pltpu.VMEM((NG,), I32),
            pltpu.VMEM((K,), I32),
            pltpu.VMEM((GKC,), I32),
            pltpu.VMEM((GKC,), I32),
            pltpu.VMEM((EKC,), I32),
        ],
    )(_sc_body)
    return run(xf).reshape(b, n)


# speedup vs baseline: 2.4893x; 1.4468x over previous
import functools

import jax
import jax.numpy as jnp
import numpy as np
from jax import lax
from jax.experimental import pallas as pl
from jax.experimental.pallas import tpu as pltpu
from jax.experimental.pallas import tpu_sc as plsc

K = 64
B = 128
N = 32768
L = 16
NW = 64
WV = 32
NG = NW * L
NSUB = 32
RPW = B // NSUB
NVEC = N // L
GKC = 1040
EKC = 2048

I32 = jnp.int32
F32 = jnp.float32
IMIN = np.int32(-2147483648)


def _ikeys(v):
    y = lax.bitcast_convert_type(v, I32)
    return jnp.where(y < 0, IMIN - y, y)


def _ivals(k):
    return lax.bitcast_convert_type(jnp.where(k < 0, IMIN - k, k), F32)


def _splat_sum_f32(v):
    tot_last = plsc.cumsum(v)
    tot_first = lax.rev(tot_last, (0,))
    first = (lax.iota(I32, L) == 0).astype(v.dtype)
    return plsc.cumsum(tot_first * first)


def _popc(m):
    return plsc.all_reduce_population_count(m)


def _bsearch(buf, nv, lo0, hi0):
    def cond(c):
        lo, hi = c
        return lo < hi

    def body(c):
        lo, hi = c
        mid = (lo & hi) + ((lo ^ hi) >> 1)
        mids = jnp.broadcast_to(mid, (L,))

        def cnt_body(j, acc):
            g = buf[pl.ds(j * L, L)]
            return acc + jnp.where(g > mids, 1, 0).astype(I32)

        cnt = jnp.sum(lax.fori_loop(0, nv, cnt_body, jnp.zeros((L,), I32)))
        go_up = cnt >= K
        return (jnp.where(go_up, mid + 1, lo), jnp.where(go_up, hi, mid))

    lo, _ = lax.while_loop(cond, body, (lo0, hi0))
    return lo


def _sc_body(x_hbm, o_hbm, row_v, gk_v, sel_v, gkc_v, gidc_v, ekc_v):
    wid = lax.axis_index("s") * 2 + lax.axis_index("c")
    lanes = lax.iota(I32, L)
    zero_i = jnp.zeros((L,), I32)

    def do_row(r, _):
        row = wid * RPW + r
        pltpu.sync_copy(x_hbm.at[row], row_v)

        racc = jnp.zeros((L,), F32)
        sgmax = []
        for j2 in range(4):

            def win_body(w2, carry, j2=j2):
                racc, w2acc = carry
                w = j2 * 16 + w2
                gmax = None
                for i in range(WV):
                    v = row_v[pl.ds((w * WV + i) * L, L)]
                    racc = racc + jnp.maximum(v, 0.0)
                    gmax = v if gmax is None else jnp.maximum(gmax, v)
                gkey = _ikeys(gmax)
                gk_v[pl.ds(w * L, L)] = gkey
                return racc, jnp.maximum(w2acc, gkey)

            racc, w2acc = lax.fori_loop(
                0, 16, win_body, (racc, jnp.full((L,), IMIN, I32))
            )
            sgmax.append(w2acc)

        mag = _splat_sum_f32(racc)
        c2 = jnp.min(jnp.minimum(jnp.minimum(sgmax[0], sgmax[1]),
                                 jnp.minimum(sgmax[2], sgmax[3])))
        hi0 = jnp.max(jnp.maximum(jnp.maximum(sgmax[0], sgmax[1]),
                                  jnp.maximum(sgmax[2], sgmax[3])))
        c2s = jnp.broadcast_to(c2, (L,))

        def compact_g(j, tot):
            g = gk_v[pl.ds(j * L, L)]
            m = g > c2s
            inc = jnp.where(m, 1, 0).astype(I32)
            pos = plsc.cumsum(inc) - 1 + tot
            plsc.store_scatter(gkc_v, [pos], g, mask=m)
            plsc.store_scatter(gidc_v, [pos], j * L + lanes, mask=m)
            return tot + _popc(m)

        totg = lax.fori_loop(0, NW, compact_g, zero_i)
        plsc.store_scatter(gkc_v, [totg + lanes], jnp.full((L,), IMIN, I32))
        totgs = jnp.max(totg)
        nvg = (totgs + 15) >> 4

        ck = _bsearch(gkc_v, nvg, c2, hi0)
        cks = jnp.broadcast_to(ck, (L,))

        for j in range(4):
            sel_v[pl.ds(j * L, L)] = zero_i

        def strictc(j, tot):
            g = gkc_v[pl.ds(j * L, L)]
            gid = gidc_v[pl.ds(j * L, L)]
            m = g > cks
            inc = jnp.where(m, 1, 0).astype(I32)
            pos = plsc.cumsum(inc) - 1 + tot
            plsc.store_scatter(sel_v, [pos], gid, mask=m)
            return tot + _popc(m)

        n_strict = lax.fori_loop(0, nvg, strictc, zero_i)

        q0 = zero_i
        for sv in range(4):
            g = sel_v[pl.ds(sv * L, L)]
            w = lax.shift_right_logical(g, 4)
            l = lax.bitwise_and(g, 15)
            gbase = w * (WV * L) + l
            valid = (sv * L + lanes) < n_strict
            for i in range(WV):
                vals = plsc.load_gather(row_v, [gbase + i * L])
                kk = _ikeys(vals)
                m = (kk > cks) & valid
                inc = jnp.where(m, 1, 0).astype(I32)
                pos = plsc.cumsum(inc) - 1 + q0
                plsc.store_scatter(ekc_v, [pos], kk, mask=m)
                q0 = q0 + _popc(m)

        plsc.store_scatter(ekc_v, [q0 + lanes], jnp.full((L,), IMIN, I32))
        q0s = jnp.max(q0)
        nvq = (q0s + 15) >> 4

        tk = _bsearch(ekc_v, nvq, ck, hi0)
        tks = jnp.broadcast_to(tk, (L,))

        def sum_body(j, accs):
            sacc, qacc = accs
            kk = ekc_v[pl.ds(j * L, L)]
            above = kk > tks
            sacc = sacc + jnp.where(above, _ivals(kk), 0.0)
            qacc = qacc + jnp.where(above, 1.0, 0.0)
            return sacc, qacc

        sacc, qacc = lax.fori_loop(
            0, nvq, sum_body, (jnp.zeros((L,), F32), jnp.zeros((L,), F32))
        )
        s_above = _splat_sum_f32(sacc)
        q = _splat_sum_f32(qacc)
        t = _ivals(tks)
        magk = s_above + t * (F32(K) - q)
        scale = jnp.where(mag > 0.0, magk / mag, 0.0)

        def out_body(j, _unused):
            for i in range(L):
                off = (j * L + i) * L
                v = row_v[pl.ds(off, L)]
                row_v[pl.ds(off, L)] = jnp.maximum(v, 0.0) * scale
            return 0

        lax.fori_loop(0, NVEC // L, out_body, 0)
        pltpu.sync_copy(row_v, o_hbm.at[row])
        return 0

    lax.fori_loop(0, RPW, do_row, 0)


@jax.jit
def kernel(x):
    b, n = x.shape
    mesh = plsc.VectorSubcoreMesh(core_axis_name="c", subcore_axis_name="s")
    run = functools.partial(
        pl.kernel,
        mesh=mesh,
        compiler_params=pltpu.CompilerParams(needs_layout_passes=False),
        out_type=jax.ShapeDtypeStruct((b, n), F32),
        scratch_types=[
            pltpu.VMEM((N,), F32),
            pltpu.VMEM((NG,), I32),
            pltpu.VMEM((K,), I32),
            pltpu.VMEM((GKC,), I32),
            pltpu.VMEM((GKC,), I32),
            pltpu.VMEM((EKC,), I32),
        ],
    )(_sc_body)
    return run(x)
